# baseline (speedup 1.0000x reference)
import functools
import math

import jax
import jax.numpy as jnp
from jax import lax
from jax.experimental import pallas as pl
from jax.experimental.pallas import tpu as pltpu
from jax.experimental.pallas import tpu_sc as plsc

D_MODEL = 64
SCALE = math.sqrt(D_MODEL)
NUM_CORES = 2
NUM_SUBCORES = 16
NUM_WORKERS = NUM_CORES * NUM_SUBCORES

CHUNK = 128
NBUF = 4
AHEAD = NBUF - 1


def _gather(tokens2_flat, table2, B):
    mesh = plsc.VectorSubcoreMesh(core_axis_name="c", subcore_axis_name="s")
    b_per_w = B // NUM_WORKERS
    n_chunks = b_per_w // CHUNK

    @functools.partial(
        pl.kernel,
        out_type=jax.ShapeDtypeStruct((B, D_MODEL), jnp.float32),
        mesh=mesh,
        scratch_types=[
            pltpu.VMEM((NBUF, CHUNK), jnp.int32),
            pltpu.VMEM((NBUF, CHUNK, D_MODEL), jnp.float32),
        ]
        + [pltpu.SemaphoreType.DMA] * (2 * NBUF),
        compiler_params=pltpu.CompilerParams(
            use_tc_tiling_on_sc=False, needs_layout_passes=False
        ),
    )
    def body(tok_hbm, tab_hbm, out_hbm, idx_v, rows_v, *sems):
        gsem = sems[:NBUF]
        ssem = sems[NBUF:]
        wid = lax.axis_index("s") * NUM_CORES + lax.axis_index("c")
        base = wid * b_per_w

        def issue_gather(g, slot):
            off = base + g * CHUNK
            pltpu.sync_copy(tok_hbm.at[pl.ds(off, CHUNK)], idx_v.at[slot])
            pltpu.async_copy(
                tab_hbm.at[idx_v.at[slot]], rows_v.at[slot], gsem[slot]
            )

        for g in range(AHEAD):
            issue_gather(g, g % NBUF)

        def outer(t, carry):
            for j in range(NBUF):
                g = t * NBUF + j
                pltpu.make_async_copy(
                    tab_hbm.at[idx_v.at[j]], rows_v.at[j], gsem[j]
                ).wait()

                off = base + g * CHUNK
                pltpu.async_copy(
                    rows_v.at[j], out_hbm.at[pl.ds(off, CHUNK)], ssem[j]
                )

                nxt = g + AHEAD

                @pl.when(nxt < n_chunks)
                def _():
                    slot = (j + AHEAD) % NBUF
                    @pl.when(nxt >= NBUF)
                    def _():
                        pltpu.make_async_copy(
                            rows_v.at[slot],
                            out_hbm.at[pl.ds(0, CHUNK)],
                            ssem[slot],
                        ).wait()

                    issue_gather(nxt, slot)

            return carry

        lax.fori_loop(0, n_chunks // NBUF, outer, 0)

        for j in range(NBUF):
            pltpu.make_async_copy(
                rows_v.at[j], out_hbm.at[pl.ds(0, CHUNK)], ssem[j]
            ).wait()

    return body(tokens2_flat, table2)


def kernel(tokens, table):
    S, SEQ = tokens.shape
    V, D = table.shape
    B = S * SEQ
    tok2 = (tokens.reshape(B) * 2).astype(jnp.int32)
    table2 = jnp.pad(table, ((0, 0), (0, 128 - D))).reshape(2 * V, D)
    out = _gather(tok2, table2, B)
    return (out * SCALE).reshape(S, SEQ, D_MODEL)


# speedup vs baseline: 1.1283x; 1.1283x over previous
import functools
import math

import jax
import jax.numpy as jnp
from jax import lax
from jax.experimental import pallas as pl
from jax.experimental.pallas import tpu as pltpu
from jax.experimental.pallas import tpu_sc as plsc

D_MODEL = 64
SCALE = math.sqrt(D_MODEL)
NUM_CORES = 2
NUM_SUBCORES = 16
NUM_WORKERS = NUM_CORES * NUM_SUBCORES

CHUNK = 128
NBUF = 4
AHEAD = NBUF - 1


def _gather(tokens_flat, table, B):
    mesh = plsc.VectorSubcoreMesh(core_axis_name="c", subcore_axis_name="s")
    b_per_w = B // NUM_WORKERS
    n_chunks = b_per_w // CHUNK

    @functools.partial(
        pl.kernel,
        out_type=jax.ShapeDtypeStruct((B, D_MODEL), jnp.float32),
        mesh=mesh,
        scratch_types=[
            pltpu.VMEM((NBUF, CHUNK), jnp.int32),
            pltpu.VMEM((NBUF, CHUNK, D_MODEL), jnp.float32),
        ]
        + [pltpu.SemaphoreType.DMA] * (2 * NBUF),
        compiler_params=pltpu.CompilerParams(
            use_tc_tiling_on_sc=False, needs_layout_passes=False
        ),
    )
    def body(tok_hbm, tab_hbm, out_hbm, idx_v, rows_v, *sems):
        gsem = sems[:NBUF]
        ssem = sems[NBUF:]
        wid = lax.axis_index("s") * NUM_CORES + lax.axis_index("c")
        base = wid * b_per_w

        def issue_gather(g, slot):
            off = base + g * CHUNK
            pltpu.sync_copy(tok_hbm.at[pl.ds(off, CHUNK)], idx_v.at[slot])
            pltpu.async_copy(
                tab_hbm.at[idx_v.at[slot]], rows_v.at[slot], gsem[slot]
            )

        for g in range(AHEAD):
            issue_gather(g, g % NBUF)

        def outer(t, carry):
            for j in range(NBUF):
                g = t * NBUF + j
                pltpu.make_async_copy(
                    tab_hbm.at[idx_v.at[j]], rows_v.at[j], gsem[j]
                ).wait()

                @plsc.parallel_loop(0, CHUNK, 1, unroll=4)
                def _(r):
                    for q in range(D_MODEL // 16):
                        sl = pl.ds(q * 16, 16)
                        rows_v[j, r, sl] = rows_v[j, r, sl] * SCALE

                off = base + g * CHUNK
                pltpu.async_copy(
                    rows_v.at[j], out_hbm.at[pl.ds(off, CHUNK)], ssem[j]
                )

                nxt = g + AHEAD

                @pl.when(nxt < n_chunks)
                def _():
                    slot = (j + AHEAD) % NBUF
                    @pl.when(nxt >= NBUF)
                    def _():
                        pltpu.make_async_copy(
                            rows_v.at[slot],
                            out_hbm.at[pl.ds(0, CHUNK)],
                            ssem[slot],
                        ).wait()

                    issue_gather(nxt, slot)

            return carry

        lax.fori_loop(0, n_chunks // NBUF, outer, 0)

        for j in range(NBUF):
            pltpu.make_async_copy(
                rows_v.at[j], out_hbm.at[pl.ds(0, CHUNK)], ssem[j]
            ).wait()

    return body(tokens_flat, table)


def kernel(tokens, table):
    S, SEQ = tokens.shape
    B = S * SEQ
    tok_flat = tokens.reshape(B).astype(jnp.int32)
    out = _gather(tok_flat, table, B)
    return out.reshape(S, SEQ, D_MODEL)
